# baseline (speedup 1.0000x reference)
import jax
import jax.numpy as jnp
from jax.experimental import pallas as pl
from jax.experimental.pallas import tpu as pltpu

_B = 4096
_R = 64
_NC = 8
_TILE = 256
_NT = _B // _TILE


def _prep_kernel(logc_ref, cores_ref, lognorm_ref, norm_ref):
    i = pl.program_id(0)
    cores_ref[0] = jax.nn.softplus(logc_ref[0])
    s = jnp.sum(cores_ref[0], axis=0)

    @pl.when(i == 0)
    def _():
        norm_ref[...] = s

    @pl.when(i > 0)
    def _():
        norm_ref[...] = norm_ref[...] @ s

    eye = (jax.lax.broadcasted_iota(jnp.int32, (_R, _R), 0)
           == jax.lax.broadcasted_iota(jnp.int32, (_R, _R), 1))
    tr = jnp.sum(jnp.where(eye, norm_ref[...], 0.0))
    lognorm_ref[...] = jnp.full((1, 1), jnp.log(tr), dtype=jnp.float32)


def _chain_kernel(idx_ref, cores_ref, out_ref):
    def body(b, carry):
        m0 = cores_ref[idx_ref[b, 0]]
        m1 = cores_ref[idx_ref[b, 1]]
        m2 = cores_ref[idx_ref[b, 2]]
        m3 = cores_ref[idx_ref[b, 3]]
        m4 = cores_ref[idx_ref[b, 4]]
        m5 = cores_ref[idx_ref[b, 5]]
        m6 = cores_ref[idx_ref[b, 6]]
        m7 = cores_ref[idx_ref[b, 7]]
        a = (m0 @ m1) @ (m2 @ m3)
        c = (m4 @ m5) @ (m6 @ m7)
        tr = jnp.sum(a * c.T)
        out_ref[0, 0, b] = tr
        return carry

    jax.lax.fori_loop(0, _TILE, body, 0, unroll=4)


def _log_kernel(tr_ref, lognorm_ref, out_ref):
    out_ref[...] = jnp.log(jnp.clip(tr_ref[...], 1e-12)) - lognorm_ref[0, 0]


def kernel(index, log_core_0, log_core_1, log_core_2, log_core_3,
           log_core_4, log_core_5, log_core_6, log_core_7):
    logc = jnp.stack([log_core_0, log_core_1, log_core_2, log_core_3,
                      log_core_4, log_core_5, log_core_6, log_core_7])

    cores, lognorm = pl.pallas_call(
        _prep_kernel,
        grid=(_NC,),
        in_specs=[pl.BlockSpec((1, _R, _R, _R), lambda i: (i, 0, 0, 0))],
        out_specs=(
            pl.BlockSpec((1, _R, _R, _R), lambda i: (i, 0, 0, 0)),
            pl.BlockSpec((1, 1), lambda i: (0, 0)),
        ),
        out_shape=(
            jax.ShapeDtypeStruct((_NC, _R, _R, _R), jnp.float32),
            jax.ShapeDtypeStruct((1, 1), jnp.float32),
        ),
        scratch_shapes=[pltpu.VMEM((_R, _R), jnp.float32)],
    )(logc)

    cores_flat = cores.reshape(_NC * _R, _R, _R)
    idx_flat = index + (jnp.arange(_NC, dtype=jnp.int32) * _R)[None, :]

    traces = pl.pallas_call(
        _chain_kernel,
        grid=(_NT,),
        in_specs=[
            pl.BlockSpec((_TILE, _NC), lambda i: (i, 0),
                         memory_space=pltpu.SMEM),
            pl.BlockSpec((_NC * _R, _R, _R), lambda i: (0, 0, 0)),
        ],
        out_specs=pl.BlockSpec((1, 1, _TILE), lambda i: (i, 0, 0),
                               memory_space=pltpu.SMEM),
        out_shape=jax.ShapeDtypeStruct((_NT, 1, _TILE), jnp.float32),
        compiler_params=pltpu.CompilerParams(
            dimension_semantics=("arbitrary",),
        ),
    )(idx_flat, cores_flat)
    traces = traces.reshape(_NT, _TILE)

    out = pl.pallas_call(
        _log_kernel,
        in_specs=[
            pl.BlockSpec((_NT, _TILE), lambda: (0, 0)),
            pl.BlockSpec((1, 1), lambda: (0, 0), memory_space=pltpu.SMEM),
        ],
        out_specs=pl.BlockSpec((_NT, _TILE), lambda: (0, 0)),
        out_shape=jax.ShapeDtypeStruct((_NT, _TILE), jnp.float32),
    )(traces, lognorm)

    return out.reshape(_B)


# speedup vs baseline: 5.4118x; 5.4118x over previous
import jax
import jax.numpy as jnp
from jax.experimental import pallas as pl
from jax.experimental.pallas import tpu as pltpu

_B = 4096
_R = 64
_NC = 8
_TILE = 64
_NT = _B // _TILE

_BMM_DIMS = (((2,), (1,)), ((0,), (0,)))


def _prep_kernel(logc_ref, cores_ref, corest_ref, lognorm_ref, norm_ref):
    i = pl.program_id(0)
    cores_ref[0] = jax.nn.softplus(logc_ref[0])
    corest_ref[0] = jnp.swapaxes(cores_ref[0], 1, 2)
    s = jnp.sum(cores_ref[0], axis=0)

    @pl.when(i == 0)
    def _():
        norm_ref[...] = s

    @pl.when(i > 0)
    def _():
        norm_ref[...] = norm_ref[...] @ s

    eye = (jax.lax.broadcasted_iota(jnp.int32, (_R, _R), 0)
           == jax.lax.broadcasted_iota(jnp.int32, (_R, _R), 1))
    tr = jnp.sum(jnp.where(eye, norm_ref[...], 0.0))
    lognorm_ref[...] = jnp.full((1, 1), jnp.log(tr), dtype=jnp.float32)


def _chain_kernel(idx_ref, cores_ref, corest_ref, lognorm_ref, out_ref,
                  g0, g1, g2, g3, g4, g5, g6, g7, s0, s1, s2, s3):
    def gather_body(b, carry):
        g0[b] = cores_ref[idx_ref[b, 0]]
        g1[b] = cores_ref[idx_ref[b, 1]]
        g2[b] = cores_ref[idx_ref[b, 2]]
        g3[b] = cores_ref[idx_ref[b, 3]]
        g4[b] = corest_ref[idx_ref[b, 4]]
        g5[b] = corest_ref[idx_ref[b, 5]]
        g6[b] = corest_ref[idx_ref[b, 6]]
        g7[b] = corest_ref[idx_ref[b, 7]]
        return carry

    jax.lax.fori_loop(0, _TILE, gather_body, 0, unroll=4)

    def bmm(x, y):
        return jax.lax.dot_general(x, y, _BMM_DIMS,
                                   preferred_element_type=jnp.float32)

    s0[...] = bmm(g0[...], g1[...])
    s1[...] = bmm(g2[...], g3[...])
    s2[...] = bmm(g7[...], g6[...])
    s3[...] = bmm(g5[...], g4[...])
    g0[...] = bmm(s0[...], s1[...])
    g1[...] = bmm(s2[...], s3[...])
    tr = jnp.sum(g0[...] * g1[...], axis=(1, 2))
    out = jnp.log(jnp.clip(tr, 1e-12)) - lognorm_ref[0, 0]
    out_ref[...] = out.reshape(1, 1, _TILE)


def kernel(index, log_core_0, log_core_1, log_core_2, log_core_3,
           log_core_4, log_core_5, log_core_6, log_core_7):
    logc = jnp.stack([log_core_0, log_core_1, log_core_2, log_core_3,
                      log_core_4, log_core_5, log_core_6, log_core_7])

    cores, cores_t, lognorm = pl.pallas_call(
        _prep_kernel,
        grid=(_NC,),
        in_specs=[pl.BlockSpec((1, _R, _R, _R), lambda i: (i, 0, 0, 0))],
        out_specs=(
            pl.BlockSpec((1, _R, _R, _R), lambda i: (i, 0, 0, 0)),
            pl.BlockSpec((1, _R, _R, _R),
                         lambda i: (jnp.maximum(i - 4, 0), 0, 0, 0)),
            pl.BlockSpec((1, 1), lambda i: (0, 0)),
        ),
        out_shape=(
            jax.ShapeDtypeStruct((_NC, _R, _R, _R), jnp.float32),
            jax.ShapeDtypeStruct((4, _R, _R, _R), jnp.float32),
            jax.ShapeDtypeStruct((1, 1), jnp.float32),
        ),
        scratch_shapes=[pltpu.VMEM((_R, _R), jnp.float32)],
    )(logc)

    cores_lo = cores.reshape(_NC * _R, _R, _R)[:4 * _R]
    corest_flat = cores_t.reshape(4 * _R, _R, _R)
    offs = jnp.array([0, 64, 128, 192, 0, 64, 128, 192], dtype=jnp.int32)
    idx_flat = index + offs[None, :]

    mscratch = [pltpu.VMEM((_TILE, _R, _R), jnp.float32) for _ in range(12)]

    out = pl.pallas_call(
        _chain_kernel,
        grid=(_NT,),
        in_specs=[
            pl.BlockSpec((_TILE, _NC), lambda i: (i, 0),
                         memory_space=pltpu.SMEM),
            pl.BlockSpec((4 * _R, _R, _R), lambda i: (0, 0, 0)),
            pl.BlockSpec((4 * _R, _R, _R), lambda i: (0, 0, 0)),
            pl.BlockSpec((1, 1), lambda i: (0, 0), memory_space=pltpu.SMEM),
        ],
        out_specs=pl.BlockSpec((1, 1, _TILE), lambda i: (i, 0, 0)),
        out_shape=jax.ShapeDtypeStruct((_NT, 1, _TILE), jnp.float32),
        scratch_shapes=mscratch,
        compiler_params=pltpu.CompilerParams(
            dimension_semantics=("arbitrary",),
        ),
    )(idx_flat, cores_lo, corest_flat, lognorm)

    return out.reshape(_B)


# speedup vs baseline: 5.4754x; 1.0117x over previous
import jax
import jax.numpy as jnp
from jax.experimental import pallas as pl
from jax.experimental.pallas import tpu as pltpu

_B = 4096
_R = 64
_NC = 8
_TILE = 64
_SUB = 4
_STEP = _TILE * _SUB
_NT = _B // _STEP

_BMM_DIMS = (((2,), (1,)), ((0,), (0,)))


def _prep_kernel(logc_ref, cores_ref, corest_ref, lognorm_ref, norm_ref):
    i = pl.program_id(0)
    cores_ref[0] = jax.nn.softplus(logc_ref[0])

    @pl.when(i >= 4)
    def _():
        corest_ref[0] = jnp.swapaxes(cores_ref[0], 1, 2)

    s = jnp.sum(cores_ref[0], axis=0)

    @pl.when(i == 0)
    def _():
        norm_ref[...] = s

    @pl.when(i > 0)
    def _():
        norm_ref[...] = norm_ref[...] @ s

    eye = (jax.lax.broadcasted_iota(jnp.int32, (_R, _R), 0)
           == jax.lax.broadcasted_iota(jnp.int32, (_R, _R), 1))
    tr = jnp.sum(jnp.where(eye, norm_ref[...], 0.0))
    lognorm_ref[...] = jnp.full((1, 1), jnp.log(tr), dtype=jnp.float32)


def _chain_kernel(idx_ref, cores_ref, corest_ref, lognorm_ref, out_ref,
                  g0, g1, g2, g3, g4, g5, g6, g7, s0, s1, s2, s3):
    def bmm(x, y):
        return jax.lax.dot_general(x, y, _BMM_DIMS,
                                   preferred_element_type=jnp.float32)

    for sub in range(_SUB):
        base = sub * _TILE

        def gather_body(b, carry):
            r = base + b
            g0[b] = cores_ref[idx_ref[r, 0]]
            g1[b] = cores_ref[idx_ref[r, 1]]
            g2[b] = cores_ref[idx_ref[r, 2]]
            g3[b] = cores_ref[idx_ref[r, 3]]
            g4[b] = corest_ref[idx_ref[r, 4]]
            g5[b] = corest_ref[idx_ref[r, 5]]
            g6[b] = corest_ref[idx_ref[r, 6]]
            g7[b] = corest_ref[idx_ref[r, 7]]
            return carry

        jax.lax.fori_loop(0, _TILE, gather_body, 0, unroll=4)

        s0[...] = bmm(g0[...], g1[...])
        s1[...] = bmm(g2[...], g3[...])
        s2[...] = bmm(g7[...], g6[...])
        s3[...] = bmm(g5[...], g4[...])
        g0[...] = bmm(s0[...], s1[...])
        g1[...] = bmm(s2[...], s3[...])
        tr = jnp.sum(g0[...] * g1[...], axis=(1, 2))
        out = jnp.log(jnp.clip(tr, 1e-12)) - lognorm_ref[0, 0]
        out_ref[0, sub] = out


def kernel(index, log_core_0, log_core_1, log_core_2, log_core_3,
           log_core_4, log_core_5, log_core_6, log_core_7):
    logc = jnp.stack([log_core_0, log_core_1, log_core_2, log_core_3,
                      log_core_4, log_core_5, log_core_6, log_core_7])

    cores, cores_t, lognorm = pl.pallas_call(
        _prep_kernel,
        grid=(_NC,),
        in_specs=[pl.BlockSpec((1, _R, _R, _R), lambda i: (i, 0, 0, 0))],
        out_specs=(
            pl.BlockSpec((1, _R, _R, _R), lambda i: (i, 0, 0, 0)),
            pl.BlockSpec((1, _R, _R, _R),
                         lambda i: (jnp.maximum(i - 4, 0), 0, 0, 0)),
            pl.BlockSpec((1, 1), lambda i: (0, 0)),
        ),
        out_shape=(
            jax.ShapeDtypeStruct((_NC, _R, _R, _R), jnp.float32),
            jax.ShapeDtypeStruct((4, _R, _R, _R), jnp.float32),
            jax.ShapeDtypeStruct((1, 1), jnp.float32),
        ),
        scratch_shapes=[pltpu.VMEM((_R, _R), jnp.float32)],
    )(logc)

    cores_lo = cores.reshape(_NC * _R, _R, _R)[:4 * _R]
    corest_flat = cores_t.reshape(4 * _R, _R, _R)
    offs = jnp.array([0, 64, 128, 192, 0, 64, 128, 192], dtype=jnp.int32)
    idx_flat = index + offs[None, :]

    mscratch = [pltpu.VMEM((_TILE, _R, _R), jnp.float32) for _ in range(12)]

    out = pl.pallas_call(
        _chain_kernel,
        grid=(_NT,),
        in_specs=[
            pl.BlockSpec((_STEP, _NC), lambda i: (i, 0),
                         memory_space=pltpu.SMEM),
            pl.BlockSpec((4 * _R, _R, _R), lambda i: (0, 0, 0)),
            pl.BlockSpec((4 * _R, _R, _R), lambda i: (0, 0, 0)),
            pl.BlockSpec((1, 1), lambda i: (0, 0), memory_space=pltpu.SMEM),
        ],
        out_specs=pl.BlockSpec((1, _SUB, _TILE), lambda i: (i, 0, 0)),
        out_shape=jax.ShapeDtypeStruct((_NT, _SUB, _TILE), jnp.float32),
        scratch_shapes=mscratch,
        compiler_params=pltpu.CompilerParams(
            dimension_semantics=("arbitrary",),
        ),
    )(idx_flat, cores_lo, corest_flat, lognorm)

    return out.reshape(_B)
